# Initial kernel scaffold; baseline (speedup 1.0000x reference)
import jax
import jax.numpy as jnp
from jax.experimental import pallas as pl


def kernel(x, ylm0, ylm1, edge_index, edge_attr, pbc_index, rev, params):
    raise NotImplementedError("write your pallas kernel here")


# speedup vs baseline: 1.1070x; 1.1070x over previous
import functools

import jax
import jax.numpy as jnp
from jax import lax
from jax.experimental import pallas as pl
from jax.experimental.pallas import tpu as pltpu

N = 10000
E = 160000
C = 64
DE = 128
DY = 64
L = 5
YIN = 25
NK = 64

BE = 2000
BN = 2000


def _hswish(x):
    return x * jnp.clip(x + 3.0, 0.0, 6.0) / 6.0


def _ylm_body(y0_ref, y1_ref, wc_ref, bc_ref, wl_ref, bl_ref, o_ref):
    def path(x):
        y = jnp.tanh(jnp.dot(x, wc_ref[...], preferred_element_type=jnp.float32)
                     + bc_ref[...])
        h1 = jnp.maximum(jnp.maximum(y[:, 0:64], y[:, 64:128]),
                         jnp.maximum(y[:, 128:192], y[:, 192:256]))
        h2 = jnp.maximum(y[:, 256:320], jnp.maximum(y[:, 320:384], y[:, 384:448]))
        h3 = jnp.maximum(y[:, 448:512], y[:, 512:576])
        h = jnp.concatenate([h1, h2, h3], axis=1)
        return jnp.dot(h, wl_ref[...], preferred_element_type=jnp.float32) + bl_ref[...]

    o_ref[:, 0:32] = path(y0_ref[...])
    o_ref[:, 32:64] = path(y1_ref[...])


def _ylm_conv(ylm0, ylm1, conv_w, conv_b, lin_w, lin_b):
    cols, bs = [], []
    for idx, k, npos in ((0, 2, 4), (1, 3, 3), (2, 4, 2)):
        w = conv_w[idx].reshape(NK, k * YIN).T
        for p in range(npos):
            col = jnp.zeros((L * YIN, NK), jnp.float32)
            col = lax.dynamic_update_slice(col, w, (p * YIN, 0))
            cols.append(col)
            bs.append(conv_b[idx])
    wc = jnp.concatenate(cols, axis=1)
    bc = jnp.concatenate(bs)
    wl = lin_w.T

    grid = E // BE
    return pl.pallas_call(
        _ylm_body,
        grid=(grid,),
        in_specs=[
            pl.BlockSpec((BE, L * YIN), lambda i: (i, 0)),
            pl.BlockSpec((BE, L * YIN), lambda i: (i, 0)),
            pl.BlockSpec((L * YIN, 576), lambda i: (0, 0)),
            pl.BlockSpec((576,), lambda i: (0,)),
            pl.BlockSpec((192, 32), lambda i: (0, 0)),
            pl.BlockSpec((32,), lambda i: (0,)),
        ],
        out_specs=pl.BlockSpec((BE, DY), lambda i: (i, 0)),
        out_shape=jax.ShapeDtypeStruct((E, DY), jnp.float32),
    )(ylm0.reshape(E, L * YIN), ylm1.reshape(E, L * YIN), wc, bc, wl, lin_b)


def _nodelin_body(h_ref, wl_ref, bl_ref, wr_ref, br_ref, xl_ref, xr_ref):
    h = h_ref[...]
    xl_ref[...] = jnp.dot(h, wl_ref[...], preferred_element_type=jnp.float32) + bl_ref[...]
    xr_ref[...] = jnp.dot(h, wr_ref[...], preferred_element_type=jnp.float32) + br_ref[...]


def _node_linears(h, p):
    grid = N // BN
    return pl.pallas_call(
        _nodelin_body,
        grid=(grid,),
        in_specs=[
            pl.BlockSpec((BN, C), lambda i: (i, 0)),
            pl.BlockSpec((C, C), lambda i: (0, 0)),
            pl.BlockSpec((C,), lambda i: (0,)),
            pl.BlockSpec((C, C), lambda i: (0, 0)),
            pl.BlockSpec((C,), lambda i: (0,)),
        ],
        out_specs=[
            pl.BlockSpec((BN, C), lambda i: (i, 0)),
            pl.BlockSpec((BN, C), lambda i: (i, 0)),
        ],
        out_shape=[
            jax.ShapeDtypeStruct((N, C), jnp.float32),
            jax.ShapeDtypeStruct((N, C), jnp.float32),
        ],
    )(h, p['lin_l_w'].T, p['lin_l_b'], p['lin_r_w'].T, p['lin_r_b'])


def _alpha_body(xls_ref, xrd_ref, ea_ref, ylm_ref, wea_ref, wylm_ref, bev_ref,
                att_ref, a_ref):
    ea = (jnp.dot(ea_ref[...], wea_ref[...], preferred_element_type=jnp.float32)
          + jnp.dot(ylm_ref[...], wylm_ref[...], preferred_element_type=jnp.float32)
          + bev_ref[...])
    m = xls_ref[...] + xrd_ref[...] + ea
    m = jnp.where(m >= 0, m, 0.2 * m)
    a_ref[...] = jnp.dot(m, att_ref[...], preferred_element_type=jnp.float32)


def _alpha(xl_src, xr_dst, edge_attr, ylm, p):
    wev = p['lin_ev_w'].T
    grid = E // BE
    return pl.pallas_call(
        _alpha_body,
        grid=(grid,),
        in_specs=[
            pl.BlockSpec((BE, C), lambda i: (i, 0)),
            pl.BlockSpec((BE, C), lambda i: (i, 0)),
            pl.BlockSpec((BE, DE), lambda i: (i, 0)),
            pl.BlockSpec((BE, DY), lambda i: (i, 0)),
            pl.BlockSpec((DE, C), lambda i: (0, 0)),
            pl.BlockSpec((DY, C), lambda i: (0, 0)),
            pl.BlockSpec((C,), lambda i: (0,)),
            pl.BlockSpec((C, 1), lambda i: (0, 0)),
        ],
        out_specs=pl.BlockSpec((BE, 1), lambda i: (i, 0)),
        out_shape=jax.ShapeDtypeStruct((E, 1), jnp.float32),
    )(xl_src, xr_dst, edge_attr, ylm, wev[:DE], wev[DE:], p['lin_ev_b'],
      p['att'].reshape(C, 1))


def _exmsg_body(a_ref, sm_ref, xls_ref, ex_ref, msg_ref):
    ex = jnp.exp(a_ref[...] - sm_ref[...])
    ex_ref[...] = ex
    msg_ref[...] = xls_ref[...] * ex


def _ex_msg(alpha, sm_dst, xl_src):
    grid = E // BE
    return pl.pallas_call(
        _exmsg_body,
        grid=(grid,),
        in_specs=[
            pl.BlockSpec((BE, 1), lambda i: (i, 0)),
            pl.BlockSpec((BE, 1), lambda i: (i, 0)),
            pl.BlockSpec((BE, C), lambda i: (i, 0)),
        ],
        out_specs=[
            pl.BlockSpec((BE, 1), lambda i: (i, 0)),
            pl.BlockSpec((BE, C), lambda i: (i, 0)),
        ],
        out_shape=[
            jax.ShapeDtypeStruct((E, 1), jnp.float32),
            jax.ShapeDtypeStruct((E, C), jnp.float32),
        ],
    )(alpha, sm_dst, xl_src)


def _out0_body(num_ref, s_ref, b_ref, o_ref):
    o_ref[...] = num_ref[...] / (s_ref[...] + 1e-16) + b_ref[...]


def _out0(num, s, bias):
    grid = N // BN
    return pl.pallas_call(
        _out0_body,
        grid=(grid,),
        in_specs=[
            pl.BlockSpec((BN, C), lambda i: (i, 0)),
            pl.BlockSpec((BN, 1), lambda i: (i, 0)),
            pl.BlockSpec((C,), lambda i: (0,)),
        ],
        out_specs=pl.BlockSpec((BN, C), lambda i: (i, 0)),
        out_shape=jax.ShapeDtypeStruct((N, C), jnp.float32),
    )(num, s, bias)


def _emlp_body(xi_ref, xj_ref, ea_ref, ylm_ref, wa_ref, wb_ref, wc_ref, wd_ref,
               b_ref, w2_ref, b2_ref, o_ref):
    t = (jnp.dot(xi_ref[...], wa_ref[...], preferred_element_type=jnp.float32)
         + jnp.dot(xj_ref[...], wb_ref[...], preferred_element_type=jnp.float32)
         + jnp.dot(ea_ref[...], wc_ref[...], preferred_element_type=jnp.float32)
         + jnp.dot(ylm_ref[...], wd_ref[...], preferred_element_type=jnp.float32)
         + b_ref[...])
    t = _hswish(t)
    o_ref[...] = (ea_ref[...]
                  + jnp.dot(t, w2_ref[...], preferred_element_type=jnp.float32)
                  + b2_ref[...])


def _edge_mlp(xi, xj, edge_attr, ylm, p):
    w = p['lin_e_w'].T
    grid = E // BE
    return pl.pallas_call(
        _emlp_body,
        grid=(grid,),
        in_specs=[
            pl.BlockSpec((BE, C), lambda i: (i, 0)),
            pl.BlockSpec((BE, C), lambda i: (i, 0)),
            pl.BlockSpec((BE, DE), lambda i: (i, 0)),
            pl.BlockSpec((BE, DY), lambda i: (i, 0)),
            pl.BlockSpec((C, DE), lambda i: (0, 0)),
            pl.BlockSpec((C, DE), lambda i: (0, 0)),
            pl.BlockSpec((DE, DE), lambda i: (0, 0)),
            pl.BlockSpec((DY, DE), lambda i: (0, 0)),
            pl.BlockSpec((DE,), lambda i: (0,)),
            pl.BlockSpec((DE, DE), lambda i: (0, 0)),
            pl.BlockSpec((DE,), lambda i: (0,)),
        ],
        out_specs=pl.BlockSpec((BE, DE), lambda i: (i, 0)),
        out_shape=jax.ShapeDtypeStruct((E, DE), jnp.float32),
    )(xi, xj, edge_attr, ylm, w[:C], w[C:2 * C], w[2 * C:2 * C + DE],
      w[2 * C + DE:], p['lin_e_b'], p['lin_e2_w'].T, p['lin_e2_b'])


def _head_body(xi_ref, xj_ref, ea_ref, ylm_ref, rev_ref, wa_ref, wb_ref,
               wc_ref, wd_ref, b1_ref, w2_ref, b2_ref, o_ref):
    t = (jnp.dot(xi_ref[...], wa_ref[...], preferred_element_type=jnp.float32)
         + jnp.dot(xj_ref[...], wb_ref[...], preferred_element_type=jnp.float32)
         + jnp.dot(ea_ref[...], wc_ref[...], preferred_element_type=jnp.float32)
         + jnp.dot(ylm_ref[...], wd_ref[...], preferred_element_type=jnp.float32)
         + b1_ref[...])
    t = _hswish(t)
    e = jnp.dot(t, w2_ref[...], preferred_element_type=jnp.float32) + b2_ref[...]
    o_ref[...] = jnp.where(rev_ref[...] == 0, e, 0.0)


def _head(xi, xj, edge_attr, ylm, rev32, params):
    w1 = params['lin1_w'].T
    grid = E // BE
    return pl.pallas_call(
        _head_body,
        grid=(grid,),
        in_specs=[
            pl.BlockSpec((BE, C), lambda i: (i, 0)),
            pl.BlockSpec((BE, C), lambda i: (i, 0)),
            pl.BlockSpec((BE, DE), lambda i: (i, 0)),
            pl.BlockSpec((BE, DY), lambda i: (i, 0)),
            pl.BlockSpec((BE, 1), lambda i: (i, 0)),
            pl.BlockSpec((C, 256), lambda i: (0, 0)),
            pl.BlockSpec((C, 256), lambda i: (0, 0)),
            pl.BlockSpec((DE, 256), lambda i: (0, 0)),
            pl.BlockSpec((DY, 256), lambda i: (0, 0)),
            pl.BlockSpec((256,), lambda i: (0,)),
            pl.BlockSpec((256, 1), lambda i: (0, 0)),
            pl.BlockSpec((1,), lambda i: (0,)),
        ],
        out_specs=pl.BlockSpec((BE, 1), lambda i: (i, 0)),
        out_shape=jax.ShapeDtypeStruct((E, 1), jnp.float32),
    )(xi, xj, edge_attr, ylm, rev32, w1[:C], w1[C:2 * C], w1[2 * C:2 * C + DE],
      w1[2 * C + DE:], params['lin1_b'], params['lin2_w'].T, params['lin2_b'])


def _gather_rows(table, idx):
    return table[idx]


def _segment_max(vals, idx):
    return jax.ops.segment_max(vals, idx, num_segments=N)


def _scatter_add(msg, ex, dst):
    num = jax.ops.segment_sum(msg, dst, num_segments=N)
    s = jax.ops.segment_sum(ex, dst, num_segments=N)
    return num, s


def kernel(x, ylm0, ylm1, edge_index, edge_attr, pbc_index, rev, params):
    src = edge_index[0]
    dst = edge_index[1]
    rev32 = rev.astype(jnp.int32).reshape(E, 1)

    h = _gather_rows(params['emb'], x)
    psrc = _gather_rows(pbc_index, src)
    pdst = _gather_rows(pbc_index, dst)

    ylm = _ylm_conv(ylm0, ylm1, params['conv_w'], params['conv_b'],
                    params['ylm_lin_w'], params['ylm_lin_b'])

    for p in params['layers']:
        xl, xr = _node_linears(h, p)
        xl_src = _gather_rows(xl, src)
        xr_dst = _gather_rows(xr, dst)
        alpha = _alpha(xl_src, xr_dst, edge_attr, ylm, p)
        segmax = _segment_max(alpha[:, 0], dst)
        sm_dst = _gather_rows(segmax, dst).reshape(E, 1)
        ex, msg = _ex_msg(alpha, sm_dst, xl_src)
        num, s = _scatter_add(msg, ex[:, 0], dst)
        out0 = _out0(num, s.reshape(N, 1), p['bias'])
        h = h + _gather_rows(out0, pbc_index)
        xi = _gather_rows(out0, psrc)
        xj = _gather_rows(out0, pdst)
        edge_attr = _edge_mlp(xi, xj, edge_attr, ylm, p)

    xi = _gather_rows(h, src)
    xj = _gather_rows(h, dst)
    return _head(xi, xj, edge_attr, ylm, rev32, params)
